# baseline (speedup 1.0000x reference)
import functools

import jax
import jax.numpy as jnp
from jax import lax
from jax.experimental import pallas as pl
from jax.experimental.pallas import tpu as pltpu
from jax.experimental.pallas import tpu_sc as plsc

_B, _L = 4096, 200
_D = 32
_V = 1000000
_N = _B * _L
_VMAIN = 999936
_TD_MAIN = _VMAIN // 4
_TD_ROWS = _TD_MAIN + (_V - _VMAIN)

_info = plsc.get_sparse_core_info()
_NC, _NS = _info.num_cores, _info.num_subcores
_NW = _NC * _NS
_mesh = plsc.VectorSubcoreMesh(core_axis_name="c", subcore_axis_name="s")

_K1C = 512
_K1_PER_W = 61


@functools.partial(
    pl.kernel,
    mesh=_mesh,
    out_type=jax.ShapeDtypeStruct((_TD_ROWS, 128), jnp.float32),
    scratch_types=[
        pltpu.VMEM((32, _K1C), jnp.float32),
        pltpu.VMEM((32, _K1C), jnp.float32),
        pltpu.VMEM((_K1C // 4, 128), jnp.float32),
        pltpu.VMEM((_K1C // 4, 128), jnp.float32),
        pltpu.VMEM((64, 128), jnp.float32),
        pltpu.SemaphoreType.DMA((2,)),
        pltpu.SemaphoreType.DMA((2,)),
    ],
    compiler_params=pltpu.CompilerParams(needs_layout_passes=False),
)
def _transpose_k(tt_hbm, tail_hbm, td_hbm, in0, in1, td0, td1, tl_v, isem, osem):
    wid = lax.axis_index("s") * _NC + lax.axis_index("c")
    ins = (in0, in1)
    tds = (td0, td1)
    base = wid * _K1_PER_W
    iota16 = lax.iota(jnp.int32, 16)

    def in_copy(ci, s):
        return pltpu.make_async_copy(
            tt_hbm.at[:, pl.ds((base + ci) * _K1C, _K1C)], ins[s], isem.at[s]
        )

    def out_copy(ci, s):
        return pltpu.make_async_copy(
            tds[s],
            td_hbm.at[pl.ds((base + ci) * (_K1C // 4), _K1C // 4)],
            osem.at[s],
        )

    rot = [(iota16 + t) & 15 for t in range(16)]

    def transpose_chunk(s):
        src = ins[s]
        dst = tds[s]
        kkvec = iota16 >> 2
        mcol = (iota16 & 3) * 32

        def blk(b16, carry):
            r16 = b16 * 16 + iota16
            kk16 = b16 * 4 + kkvec
            for h in (0, 16):
                colb = mcol + h
                for t in range(16):
                    vals = plsc.load_gather(src, [rot[t] + h, r16])
                    plsc.store_scatter(dst, [kk16, colb + rot[t]], vals)
            return carry

        lax.fori_loop(0, _K1C // 16, blk, 0)

    @pl.when(wid == _NW - 1)
    def _extra():
        pltpu.sync_copy(tail_hbm, tl_v)
        pltpu.sync_copy(tl_v, td_hbm.at[pl.ds(_TD_MAIN, _V - _VMAIN)])
        pltpu.sync_copy(tt_hbm.at[:, pl.ds(1952 * _K1C, _K1C)], in0)
        transpose_chunk(0)
        pltpu.sync_copy(td0, td_hbm.at[pl.ds(1952 * (_K1C // 4), _K1C // 4)])

    in_copy(0, 0).start()
    in_copy(1, 1).start()

    def pair(p, carry):
        for u in range(2):
            ci = p * 2 + u
            in_copy(ci, u).wait()

            @pl.when(ci >= 2)
            def _():
                out_copy(ci - 2, u).wait()

            transpose_chunk(u)
            out_copy(ci, u).start()
            in_copy(ci + 2, u).start()
        return carry

    lax.fori_loop(0, (_K1_PER_W - 1) // 2, pair, 0)
    in_copy(60, 0).wait()
    out_copy(58, 0).wait()
    transpose_chunk(0)
    out_copy(60, 0).start()
    in_copy(61, 1).wait()
    out_copy(59, 1).wait()
    out_copy(60, 0).wait()


_PERW = _N // _NW
_NL = 2
_CH = _NL * 128
_NCH2 = _L // _NL


@functools.partial(
    pl.kernel,
    mesh=_mesh,
    out_type=jax.ShapeDtypeStruct((_L, _D, _B), jnp.float32),
    scratch_types=[
        pltpu.VMEM((_PERW,), jnp.int32),
        pltpu.VMEM((_CH,), jnp.int32),
        pltpu.VMEM((_CH,), jnp.int32),
        pltpu.VMEM((_CH, 128), jnp.float32),
        pltpu.VMEM((_CH, 128), jnp.float32),
        pltpu.VMEM((_NL, _D, 128), jnp.float32),
        pltpu.VMEM((_NL, _D, 128), jnp.float32),
        pltpu.SemaphoreType.DMA,
        pltpu.SemaphoreType.DMA((2,)),
        pltpu.SemaphoreType.DMA((2,)),
    ],
    compiler_params=pltpu.CompilerParams(needs_layout_passes=False),
)
def _gather_k(idx_hbm, td_hbm, out_hbm, idx_v, q0, q1, g0, g1, o0, o1,
              stsem, gsem, osem):
    wid = lax.axis_index("s") * _NC + lax.axis_index("c")
    b0 = wid * 128
    qs = (q0, q1)
    gs = (g0, g1)
    ov = (o0, o1)
    iota16 = lax.iota(jnp.int32, 16)

    pltpu.async_copy(idx_hbm.at[pl.ds(b0 * _L, _PERW)], idx_v, stsem).wait()

    def build_q(i, s):
        q = qs[s]

        def grp(g, carry):
            r = g // 8
            bb = (g % 8) * 16
            src = (bb + iota16) * _L + (i * _NL + r)
            v = plsc.load_gather(idx_v, [src])
            tail = v >= _VMAIN
            qv = jnp.where(tail, v - _VMAIN + _TD_MAIN, v >> 2)
            q[pl.ds(r * 128 + bb, 16)] = qv
            return carry

        lax.fori_loop(0, _NL * 8, grp, 0)

    def gather_copy(s):
        return pltpu.make_async_copy(td_hbm.at[qs[s]], gs[s], gsem.at[s])

    def out_copy(i, s):
        return pltpu.make_async_copy(
            ov[s],
            out_hbm.at[pl.ds(i * _NL, _NL), :, pl.ds(b0, 128)],
            osem.at[s],
        )

    rot = [(iota16 + t) & 15 for t in range(16)]

    def select(i, s):
        g = gs[s]
        o = ov[s]

        def grp(k, carry):
            r = k // 8
            bb = (k % 8) * 16
            src = (bb + iota16) * _L + (i * _NL + r)
            v = plsc.load_gather(idx_v, [src])
            m = jnp.where(v >= _VMAIN, 0, v & 3)
            row16 = jnp.full((16,), r * 128 + bb, jnp.int32) + iota16
            b16 = jnp.full((16,), bb, jnp.int32) + iota16
            r16 = jnp.full((16,), r, jnp.int32)
            for h in (0, 16):
                colb = m * 32 + h
                for t in range(16):
                    vals = plsc.load_gather(g, [row16, colb + rot[t]])
                    plsc.store_scatter(o, [r16, rot[t] + h, b16], vals)
            return carry

        lax.fori_loop(0, _NL * 8, grp, 0)

    build_q(0, 0)
    gather_copy(0).start()

    def pair(p, carry):
        for u in range(2):
            i = p * 2 + u + 1
            s = (u + 1) % 2

            @pl.when(i < _NCH2)
            def _():
                build_q(i, s)
                gather_copy(s).start()

            ip = i - 1
            sp = u % 2

            @pl.when(ip >= 2)
            def _():
                out_copy(ip - 2, sp).wait()

            gather_copy(sp).wait()
            select(ip, sp)
            out_copy(ip, sp).start()
        return carry

    lax.fori_loop(0, _NCH2 // 2, pair, 0)
    out_copy(_NCH2 - 2, 0).wait()
    out_copy(_NCH2 - 1, 1).wait()


@jax.jit
def kernel(x, table):
    idx = x.reshape(-1).astype(jnp.int32)
    tt = table.T
    tail = jnp.pad(
        lax.slice(table, (_VMAIN, 0), (_V, _D)), ((0, 0), (0, 128 - _D))
    )
    td = _transpose_k(tt, tail)
    out_t = _gather_k(idx, td)
    return out_t.transpose(2, 0, 1)


# speedup vs baseline: 1.5189x; 1.5189x over previous
import functools

import jax
import jax.numpy as jnp
from jax import lax
from jax.experimental import pallas as pl
from jax.experimental.pallas import tpu as pltpu
from jax.experimental.pallas import tpu_sc as plsc

_B, _L = 4096, 200
_D = 32
_V = 1000000
_N = _B * _L
_VMAIN = 999936
_TD_MAIN = _VMAIN // 4
_TD_ROWS = _TD_MAIN + (_V - _VMAIN)

_info = plsc.get_sparse_core_info()
_NC, _NS = _info.num_cores, _info.num_subcores
_NW = _NC * _NS
_mesh = plsc.VectorSubcoreMesh(core_axis_name="c", subcore_axis_name="s")

_K1C = 512
_K1_PER_W = 61


@functools.partial(
    pl.kernel,
    mesh=_mesh,
    out_type=jax.ShapeDtypeStruct((_TD_ROWS, 128), jnp.float32),
    scratch_types=[
        pltpu.VMEM((32, _K1C), jnp.float32),
        pltpu.VMEM((32, _K1C), jnp.float32),
        pltpu.VMEM((_K1C // 4, 128), jnp.float32),
        pltpu.VMEM((_K1C // 4, 128), jnp.float32),
        pltpu.VMEM((64, 128), jnp.float32),
        pltpu.SemaphoreType.DMA((2,)),
        pltpu.SemaphoreType.DMA((2,)),
    ],
    compiler_params=pltpu.CompilerParams(needs_layout_passes=False),
)
def _transpose_k(tt_hbm, tail_hbm, td_hbm, in0, in1, td0, td1, tl_v, isem, osem):
    wid = lax.axis_index("s") * _NC + lax.axis_index("c")
    ins = (in0, in1)
    tds = (td0, td1)
    base = wid * _K1_PER_W
    iota16 = lax.iota(jnp.int32, 16)

    def in_copy(ci, s):
        return pltpu.make_async_copy(
            tt_hbm.at[:, pl.ds((base + ci) * _K1C, _K1C)], ins[s], isem.at[s]
        )

    def out_copy(ci, s):
        return pltpu.make_async_copy(
            tds[s],
            td_hbm.at[pl.ds((base + ci) * (_K1C // 4), _K1C // 4)],
            osem.at[s],
        )

    rot = [(iota16 + t) & 15 for t in range(16)]

    def transpose_chunk(s):
        src = ins[s]
        dst = tds[s]
        kkvec = iota16 >> 2
        mcol = (iota16 & 3) * 32

        @plsc.parallel_loop(0, _K1C // 16, unroll=2)
        def _blk(b16):
            r16 = b16 * 16 + iota16
            kk16 = b16 * 4 + kkvec
            for h in (0, 16):
                colb = mcol + h
                for t in range(16):
                    vals = plsc.load_gather(src, [rot[t] + h, r16])
                    plsc.store_scatter(dst, [kk16, colb + rot[t]], vals)

    @pl.when(wid == _NW - 1)
    def _extra():
        pltpu.sync_copy(tail_hbm, tl_v)
        pltpu.sync_copy(tl_v, td_hbm.at[pl.ds(_TD_MAIN, _V - _VMAIN)])
        pltpu.sync_copy(tt_hbm.at[:, pl.ds(1952 * _K1C, _K1C)], in0)
        transpose_chunk(0)
        pltpu.sync_copy(td0, td_hbm.at[pl.ds(1952 * (_K1C // 4), _K1C // 4)])

    in_copy(0, 0).start()
    in_copy(1, 1).start()

    def pair(p, carry):
        for u in range(2):
            ci = p * 2 + u
            in_copy(ci, u).wait()

            @pl.when(ci >= 2)
            def _():
                out_copy(ci - 2, u).wait()

            transpose_chunk(u)
            out_copy(ci, u).start()
            in_copy(ci + 2, u).start()
        return carry

    lax.fori_loop(0, (_K1_PER_W - 1) // 2, pair, 0)
    in_copy(60, 0).wait()
    out_copy(58, 0).wait()
    transpose_chunk(0)
    out_copy(60, 0).start()
    in_copy(61, 1).wait()
    out_copy(59, 1).wait()
    out_copy(60, 0).wait()


_PERW = _N // _NW
_NL = 2
_CH = _NL * 128
_NCH2 = _L // _NL


@functools.partial(
    pl.kernel,
    mesh=_mesh,
    out_type=jax.ShapeDtypeStruct((_L, _D, _B), jnp.float32),
    scratch_types=[
        pltpu.VMEM((_PERW,), jnp.int32),
        pltpu.VMEM((_CH,), jnp.int32),
        pltpu.VMEM((_CH,), jnp.int32),
        pltpu.VMEM((_CH, 128), jnp.float32),
        pltpu.VMEM((_CH, 128), jnp.float32),
        pltpu.VMEM((_NL, _D, 128), jnp.float32),
        pltpu.VMEM((_NL, _D, 128), jnp.float32),
        pltpu.SemaphoreType.DMA,
        pltpu.SemaphoreType.DMA((2,)),
        pltpu.SemaphoreType.DMA((2,)),
    ],
    compiler_params=pltpu.CompilerParams(needs_layout_passes=False),
)
def _gather_k(idx_hbm, td_hbm, out_hbm, idx_v, q0, q1, g0, g1, o0, o1,
              stsem, gsem, osem):
    wid = lax.axis_index("s") * _NC + lax.axis_index("c")
    b0 = wid * 128
    qs = (q0, q1)
    gs = (g0, g1)
    ov = (o0, o1)
    iota16 = lax.iota(jnp.int32, 16)

    pltpu.async_copy(idx_hbm.at[pl.ds(b0 * _L, _PERW)], idx_v, stsem).wait()

    def build_q(i, s):
        q = qs[s]

        def grp(g, carry):
            r = g // 8
            bb = (g % 8) * 16
            src = (bb + iota16) * _L + (i * _NL + r)
            v = plsc.load_gather(idx_v, [src])
            tail = v >= _VMAIN
            qv = jnp.where(tail, v - _VMAIN + _TD_MAIN, v >> 2)
            q[pl.ds(r * 128 + bb, 16)] = qv
            return carry

        lax.fori_loop(0, _NL * 8, grp, 0)

    def gather_copy(s):
        return pltpu.make_async_copy(td_hbm.at[qs[s]], gs[s], gsem.at[s])

    def out_copy(i, s):
        return pltpu.make_async_copy(
            ov[s],
            out_hbm.at[pl.ds(i * _NL, _NL), :, pl.ds(b0, 128)],
            osem.at[s],
        )

    rot = [(iota16 + t) & 15 for t in range(16)]

    def select(i, s):
        g = gs[s]
        o = ov[s]

        @plsc.parallel_loop(0, _NL * 8, unroll=2)
        def _grp(k):
            r = k // 8
            bb = (k % 8) * 16
            src = (bb + iota16) * _L + (i * _NL + r)
            v = plsc.load_gather(idx_v, [src])
            m = jnp.where(v >= _VMAIN, 0, v & 3)
            row16 = jnp.full((16,), r * 128 + bb, jnp.int32) + iota16
            b16 = jnp.full((16,), bb, jnp.int32) + iota16
            r16 = jnp.full((16,), r, jnp.int32)
            for h in (0, 16):
                colb = m * 32 + h
                for t in range(16):
                    vals = plsc.load_gather(g, [row16, colb + rot[t]])
                    plsc.store_scatter(o, [r16, rot[t] + h, b16], vals)

    build_q(0, 0)
    gather_copy(0).start()

    def pair(p, carry):
        for u in range(2):
            i = p * 2 + u + 1
            s = (u + 1) % 2

            @pl.when(i < _NCH2)
            def _():
                build_q(i, s)
                gather_copy(s).start()

            ip = i - 1
            sp = u % 2

            @pl.when(ip >= 2)
            def _():
                out_copy(ip - 2, sp).wait()

            gather_copy(sp).wait()
            select(ip, sp)
            out_copy(ip, sp).start()
        return carry

    lax.fori_loop(0, _NCH2 // 2, pair, 0)
    out_copy(_NCH2 - 2, 0).wait()
    out_copy(_NCH2 - 1, 1).wait()


@jax.jit
def kernel(x, table):
    idx = x.reshape(-1).astype(jnp.int32)
    tt = table.T
    tail = jnp.pad(
        lax.slice(table, (_VMAIN, 0), (_V, _D)), ((0, 0), (0, 128 - _D))
    )
    td = _transpose_k(tt, tail)
    out_t = _gather_k(idx, td)
    return out_t.transpose(2, 0, 1)
